# baseline (speedup 1.0000x reference)
import functools

import jax
import jax.numpy as jnp
from jax import lax
from jax.experimental import pallas as pl
from jax.experimental.pallas import tpu as pltpu
from jax.experimental.pallas import tpu_sc as plsc

_B, _L, _E = 1024, 200, 64
_N = _B * _L
_NC, _NS = 2, 16
_NW = _NC * _NS
_ROWS_W = _N // _NW
_SEQ_W = _ROWS_W // _L
_HL = _L // 2
_NBUF = 4

_mesh = plsc.VectorSubcoreMesh(core_axis_name="c", subcore_axis_name="s")


@functools.partial(
    pl.kernel,
    mesh=_mesh,
    out_type=jax.ShapeDtypeStruct((_N, _E), jnp.float32),
    scratch_types=[
        pltpu.VMEM((2 * _SEQ_W, _HL), jnp.int32),
        pltpu.VMEM((2 * _SEQ_W, _HL), jnp.int32),
        [pltpu.VMEM((_L, _E), jnp.float32) for _ in range(_NBUF)],
        [pltpu.SemaphoreType.DMA for _ in range(3 * _NBUF)],
    ],
    compiler_params=pltpu.CompilerParams(use_tc_tiling_on_sc=False),
)
def _emb_kernel(seq_hbm, seglab_hbm, tok_tab, seg_tab, pe_hbm, out_hbm,
                tokidx, segidx, outb, sems):
    wid = lax.axis_index("s") * _NC + lax.axis_index("c")
    base = wid * _ROWS_W
    sem_p, sem_a, sem_sc = sems[0:_NBUF], sems[_NBUF:2 * _NBUF], sems[2 * _NBUF:]

    pltpu.sync_copy(seq_hbm.at[wid], tokidx)
    pltpu.sync_copy(seglab_hbm.at[wid], segidx)

    def prefill(s):
        b = s % _NBUF
        return pltpu.make_async_copy(pe_hbm, outb[b], sem_p[b])

    def adds(s):
        b = s % _NBUF
        out = []
        for h in range(2):
            half = outb[b].at[pl.ds(h * _HL, _HL)]
            out.append(pltpu.make_async_copy(
                tok_tab.at[tokidx.at[2 * s + h]], half, sem_a[b]))
            out.append(pltpu.make_async_copy(
                seg_tab.at[segidx.at[2 * s + h]], half, sem_a[b]))
        return out

    def scatter(s):
        b = s % _NBUF
        return pltpu.make_async_copy(
            outb[b], out_hbm.at[pl.ds(base + s * _L, _L)], sem_sc[b])

    def start_adds(s):
        for h in adds(s):
            h.start(add=True)

    def wait_adds(s):
        for h in adds(s):
            h.wait()

    for s in range(_SEQ_W):
        if s >= 2:
            wait_adds(s - 2)
            scatter(s - 2).start()
        if s >= _NBUF:
            scatter(s - _NBUF).wait()
        prefill(s).start()
        if s >= 1:
            prefill(s - 1).wait()
            start_adds(s - 1)

    prefill(_SEQ_W - 1).wait()
    start_adds(_SEQ_W - 1)
    for s in (_SEQ_W - 2, _SEQ_W - 1):
        wait_adds(s)
        scatter(s).start()
    for s in range(_SEQ_W - _NBUF, _SEQ_W):
        scatter(s).wait()


def kernel(sequence, segment_label, token_table, segment_table, pe):
    seq = sequence.reshape(_NW, 2 * _SEQ_W, _HL).astype(jnp.int32)
    seg = segment_label.reshape(_NW, 2 * _SEQ_W, _HL).astype(jnp.int32)
    pe_l = pe[0, :_L, :].astype(jnp.float32)
    out = _emb_kernel(seq, seg, token_table, segment_table, pe_l)
    return out.reshape(_B, _L, _E)


# speedup vs baseline: 2.0727x; 2.0727x over previous
import functools

import jax
import jax.numpy as jnp
from jax import lax
from jax.experimental import pallas as pl
from jax.experimental.pallas import tpu as pltpu
from jax.experimental.pallas import tpu_sc as plsc

_B, _L, _E = 1024, 200, 64
_N = _B * _L
_NSEG = 17
_NC, _NS = 2, 16
_NW = _NC * _NS
_ROWS_W = _N // _NW
_SEQ_W = _ROWS_W // _L
_HL = _L // 2
_NBUF = 4

_mesh = plsc.VectorSubcoreMesh(core_axis_name="c", subcore_axis_name="s")


@functools.partial(
    pl.kernel,
    mesh=_mesh,
    out_type=jax.ShapeDtypeStruct((_N, _E), jnp.float32),
    scratch_types=[
        pltpu.VMEM((2 * _SEQ_W, _HL), jnp.int32),
        pltpu.VMEM((_SEQ_W, _L), jnp.int32),
        pltpu.VMEM((_NSEG, _E), jnp.float32),
        [pltpu.VMEM((_L, _E), jnp.float32) for _ in range(_NBUF)],
        [pltpu.SemaphoreType.DMA for _ in range(3 * _NBUF)],
    ],
    compiler_params=pltpu.CompilerParams(use_tc_tiling_on_sc=False),
)
def _emb_kernel(seq_hbm, seglab_hbm, tok_tab, seg_tab, pe_hbm, out_hbm,
                tokidx, seglab, segtab_v, outb, sems):
    wid = lax.axis_index("s") * _NC + lax.axis_index("c")
    base = wid * _ROWS_W
    sem_p, sem_a, sem_sc = sems[0:_NBUF], sems[_NBUF:2 * _NBUF], sems[2 * _NBUF:]

    pltpu.sync_copy(seq_hbm.at[wid], tokidx)
    pltpu.sync_copy(seglab_hbm.at[wid], seglab)
    pltpu.sync_copy(seg_tab, segtab_v)

    def prefill(s, b):
        return pltpu.make_async_copy(pe_hbm, outb[b], sem_p[b])

    def adds(s, b):
        out = []
        for h in range(2):
            half = outb[b].at[pl.ds(h * _HL, _HL)]
            out.append(pltpu.make_async_copy(
                tok_tab.at[tokidx.at[2 * s + h]], half, sem_a[b]))
        return out

    def scatter(s, b):
        return pltpu.make_async_copy(
            outb[b], out_hbm.at[pl.ds(base + s * _L, _L)], sem_sc[b])

    def seg_add(s, b):
        ob = outb[b]

        def rows16(r0, labs, lo=0):
            for k in range(lo, 16):
                lab = labs[k]
                r = r0 + k
                for p2 in range(_E // 16):
                    sl = pl.ds(p2 * 16, 16)
                    ob[r, sl] = ob[r, sl] + segtab_v[lab, sl]

        def grp_body(i, c2):
            r0 = i * 16
            rows16(r0, seglab[s, pl.ds(r0, 16)])
            return c2

        lax.fori_loop(0, _L // 16, grp_body, 0)
        rows16(184, seglab[s, pl.ds(184, 16)], lo=8)

    def visit(s, known):
        b = s % 4 if isinstance(s, int) else None

        def B(expr):
            return expr % 4

        if isinstance(s, int):
            if 2 <= s:
                t = s - 2
                for h in adds(t, t % 4):
                    h.wait()
                seg_add(t, t % 4)
                scatter(t, t % 4).start()
            if 4 <= s:
                scatter(s - 4, (s - 4) % 4).wait()
            if s <= _SEQ_W - 1:
                prefill(s, s % 4).start()
            if 1 <= s:
                prefill(s - 1, (s - 1) % 4).wait()
                for h in adds(s - 1, (s - 1) % 4):
                    h.start(add=True)

    for s in range(4):
        visit(s, None)

    def steady(u, carry):
        for b in range(4):
            s = 4 * u + b
            t = s - 2
            for h in adds(t, (b - 2) % 4):
                h.wait()
            seg_add(t, (b - 2) % 4)
            scatter(t, (b - 2) % 4).start()
            scatter(s - 4, b).wait()
            prefill(s, b).start()
            prefill(s - 1, (b - 1) % 4).wait()
            for h in adds(s - 1, (b - 1) % 4):
                h.start(add=True)
        return carry

    lax.fori_loop(1, _SEQ_W // 4, steady, 0)

    for s in (_SEQ_W, _SEQ_W + 1):
        t = s - 2
        for h in adds(t, t % 4):
            h.wait()
        seg_add(t, t % 4)
        scatter(t, t % 4).start()
        scatter(s - 4, (s - 4) % 4).wait()
        if s == _SEQ_W:
            prefill(s - 1, (s - 1) % 4).wait()
            for h in adds(s - 1, (s - 1) % 4):
                h.start(add=True)
    scatter(_SEQ_W - 2, (_SEQ_W - 2) % 4).wait()
    scatter(_SEQ_W - 1, (_SEQ_W - 1) % 4).wait()


def kernel(sequence, segment_label, token_table, segment_table, pe):
    seq = sequence.reshape(_NW, 2 * _SEQ_W, _HL).astype(jnp.int32)
    seg = segment_label.reshape(_NW, _SEQ_W, _L).astype(jnp.int32)
    pe_l = pe[0, :_L, :].astype(jnp.float32)
    out = _emb_kernel(seq, seg, token_table, segment_table, pe_l)
    return out.reshape(_B, _L, _E)


# speedup vs baseline: 2.0733x; 1.0003x over previous
import functools

import jax
import jax.numpy as jnp
from jax import lax
from jax.experimental import pallas as pl
from jax.experimental.pallas import tpu as pltpu
from jax.experimental.pallas import tpu_sc as plsc

_B, _L, _E = 1024, 200, 64
_N = _B * _L
_NSEG = 17
_NC, _NS = 2, 16
_NW = _NC * _NS
_ROWS_W = _N // _NW
_SEQ_W = _ROWS_W // _L
_HL = _L // 2
_NBUF = 4

_mesh = plsc.VectorSubcoreMesh(core_axis_name="c", subcore_axis_name="s")


@functools.partial(
    pl.kernel,
    mesh=_mesh,
    out_type=jax.ShapeDtypeStruct((_B, _L, _E), jnp.float32),
    scratch_types=[
        pltpu.VMEM((2 * _SEQ_W, _HL), jnp.int32),
        pltpu.VMEM((_SEQ_W, _L), jnp.int32),
        pltpu.VMEM((_NSEG, _E), jnp.float32),
        [pltpu.VMEM((_L, _E), jnp.float32) for _ in range(_NBUF)],
        [pltpu.SemaphoreType.DMA for _ in range(3 * _NBUF)],
    ],
    compiler_params=pltpu.CompilerParams(use_tc_tiling_on_sc=False),
)
def _emb_kernel(seq_hbm, seglab_hbm, tok_tab, seg_tab, pe_hbm, out_hbm,
                tokidx, seglab, segtab_v, outb, sems):
    wid = lax.axis_index("s") * _NC + lax.axis_index("c")
    base = wid * _ROWS_W
    sem_p, sem_a, sem_sc = sems[0:_NBUF], sems[_NBUF:2 * _NBUF], sems[2 * _NBUF:]

    pltpu.sync_copy(seq_hbm.at[wid], tokidx)
    pltpu.sync_copy(seglab_hbm.at[wid], seglab)
    pltpu.sync_copy(seg_tab, segtab_v)

    def prefill(s, b):
        return pltpu.make_async_copy(pe_hbm, outb[b], sem_p[b])

    def adds(s, b):
        out = []
        for h in range(2):
            half = outb[b].at[pl.ds(h * _HL, _HL)]
            out.append(pltpu.make_async_copy(
                tok_tab.at[tokidx.at[2 * s + h]], half, sem_a[b]))
        return out

    def scatter(s, b):
        return pltpu.make_async_copy(
            outb[b], out_hbm.at[wid * _SEQ_W + s], sem_sc[b])

    def seg_add(s, b):
        ob = outb[b]

        def rows16(r0, labs, lo=0):
            for k in range(lo, 16):
                lab = labs[k]
                r = r0 + k
                for p2 in range(_E // 16):
                    sl = pl.ds(p2 * 16, 16)
                    ob[r, sl] = ob[r, sl] + segtab_v[lab, sl]

        def grp_body(i, c2):
            r0 = i * 16
            rows16(r0, seglab[s, pl.ds(r0, 16)])
            return c2

        lax.fori_loop(0, _L // 16, grp_body, 0)
        rows16(184, seglab[s, pl.ds(184, 16)], lo=8)

    def visit(s, known):
        b = s % 4 if isinstance(s, int) else None

        def B(expr):
            return expr % 4

        if isinstance(s, int):
            if 2 <= s:
                t = s - 2
                for h in adds(t, t % 4):
                    h.wait()
                seg_add(t, t % 4)
                scatter(t, t % 4).start()
            if 4 <= s:
                scatter(s - 4, (s - 4) % 4).wait()
            if s <= _SEQ_W - 1:
                prefill(s, s % 4).start()
            if 1 <= s:
                prefill(s - 1, (s - 1) % 4).wait()
                for h in adds(s - 1, (s - 1) % 4):
                    h.start(add=True)

    for s in range(4):
        visit(s, None)

    def steady(u, carry):
        for b in range(4):
            s = 4 * u + b
            t = s - 2
            for h in adds(t, (b - 2) % 4):
                h.wait()
            seg_add(t, (b - 2) % 4)
            scatter(t, (b - 2) % 4).start()
            scatter(s - 4, b).wait()
            prefill(s, b).start()
            prefill(s - 1, (b - 1) % 4).wait()
            for h in adds(s - 1, (b - 1) % 4):
                h.start(add=True)
        return carry

    lax.fori_loop(1, _SEQ_W // 4, steady, 0)

    for s in (_SEQ_W, _SEQ_W + 1):
        t = s - 2
        for h in adds(t, t % 4):
            h.wait()
        seg_add(t, t % 4)
        scatter(t, t % 4).start()
        scatter(s - 4, (s - 4) % 4).wait()
        if s == _SEQ_W:
            prefill(s - 1, (s - 1) % 4).wait()
            for h in adds(s - 1, (s - 1) % 4):
                h.start(add=True)
    scatter(_SEQ_W - 2, (_SEQ_W - 2) % 4).wait()
    scatter(_SEQ_W - 1, (_SEQ_W - 1) % 4).wait()


def kernel(sequence, segment_label, token_table, segment_table, pe):
    seq = sequence.reshape(_NW, 2 * _SEQ_W, _HL).astype(jnp.int32)
    seg = segment_label.reshape(_NW, _SEQ_W, _L).astype(jnp.int32)
    pe_l = pe[0, :_L, :].astype(jnp.float32)
    return _emb_kernel(seq, seg, token_table, segment_table, pe_l)


# speedup vs baseline: 2.1225x; 1.0237x over previous
import functools

import jax
import jax.numpy as jnp
from jax import lax
from jax.experimental import pallas as pl
from jax.experimental.pallas import tpu as pltpu
from jax.experimental.pallas import tpu_sc as plsc

_B, _L, _E = 1024, 200, 64
_N = _B * _L
_NSEG = 17
_NC, _NS = 2, 16
_NW = _NC * _NS
_ROWS_W = _N // _NW
_SEQ_W = _ROWS_W // _L
_HL = _L // 2
_NBUF = 4

_mesh = plsc.VectorSubcoreMesh(core_axis_name="c", subcore_axis_name="s")


@functools.partial(
    pl.kernel,
    mesh=_mesh,
    out_type=jax.ShapeDtypeStruct((_B, _L, _E), jnp.float32),
    scratch_types=[
        pltpu.VMEM((2 * _SEQ_W, _HL), jnp.int32),
        pltpu.VMEM((_SEQ_W, _L), jnp.int32),
        pltpu.VMEM((_NSEG, _E), jnp.float32),
        [pltpu.VMEM((_L, _E), jnp.float32) for _ in range(_NBUF)],
        [pltpu.SemaphoreType.DMA for _ in range(3 * _NBUF)],
    ],
    compiler_params=pltpu.CompilerParams(use_tc_tiling_on_sc=False),
)
def _emb_kernel(seq_hbm, seglab_hbm, tok_tab, seg_tab, pe_hbm, out_hbm,
                tokidx, seglab, segtab_v, outb, sems):
    wid = lax.axis_index("s") * _NC + lax.axis_index("c")
    base = wid * _ROWS_W
    sem_p, sem_a, sem_sc = sems[0:_NBUF], sems[_NBUF:2 * _NBUF], sems[2 * _NBUF:]

    pltpu.sync_copy(seq_hbm.at[wid], tokidx)
    pltpu.sync_copy(seglab_hbm.at[wid], seglab)
    pltpu.sync_copy(seg_tab, segtab_v)

    def prefill(s, b):
        return pltpu.make_async_copy(pe_hbm, outb[b], sem_p[b])

    def adds(s, b):
        out = []
        for h in range(2):
            half = outb[b].at[pl.ds(h * _HL, _HL)]
            out.append(pltpu.make_async_copy(
                tok_tab.at[tokidx.at[2 * s + h]], half, sem_a[b]))
        return out

    def scatter(s, b):
        return pltpu.make_async_copy(
            outb[b], out_hbm.at[wid * _SEQ_W + s], sem_sc[b])

    def seg_add(s, b):
        ob = outb[b]

        def rows16(r0, labs, lo=0):
            for k in range(lo, 16):
                lab = labs[k]
                r = r0 + k
                for p2 in range(_E // 16):
                    sl = pl.ds(p2 * 16, 16)
                    ob[r, sl] = ob[r, sl] + segtab_v[lab, sl]

        def grp_body(i, c2):
            r0 = i * 16
            rows16(r0, seglab[s, pl.ds(r0, 16)])
            return c2

        lax.fori_loop(0, _L // 16, grp_body, 0)
        rows16(184, seglab[s, pl.ds(184, 16)], lo=8)

    def start_adds(s, b):
        for h in adds(s, b):
            h.start(add=True)

    def wait_adds(s, b):
        for h in adds(s, b):
            h.wait()

    prefill(0, 0).start()
    prefill(1, 1).start()
    prefill(0, 0).wait()
    start_adds(0, 0)
    prefill(2, 2).start()
    prefill(1, 1).wait()
    start_adds(1, 1)
    for s in (2, 3):
        t = s - 2
        wait_adds(t, t % 4)
        seg_add(t, t % 4)
        scatter(t, t % 4).start()
        if s == 3:
            scatter(0, 0).wait()
        prefill(s + 1, (s + 1) % 4).start()
        prefill(s, s % 4).wait()
        start_adds(s, s % 4)

    def steady(u, carry):
        for b in range(4):
            s = 4 * u + b
            t = s - 2
            wait_adds(t, (b - 2) % 4)
            seg_add(t, (b - 2) % 4)
            scatter(t, (b - 2) % 4).start()
            scatter(s - 3, (b + 1) % 4).wait()

            @pl.when(s < _SEQ_W - 1)
            def _():
                prefill(s + 1, (b + 1) % 4).start()

            prefill(s, b).wait()
            start_adds(s, b)
        return carry

    lax.fori_loop(1, _SEQ_W // 4, steady, 0)

    for s in (_SEQ_W, _SEQ_W + 1):
        t = s - 2
        wait_adds(t, t % 4)
        seg_add(t, t % 4)
        scatter(t, t % 4).start()
    for t in (_SEQ_W - 3, _SEQ_W - 2, _SEQ_W - 1):
        scatter(t, t % 4).wait()


def kernel(sequence, segment_label, token_table, segment_table, pe):
    seq = sequence.reshape(_NW, 2 * _SEQ_W, _HL).astype(jnp.int32)
    seg = segment_label.reshape(_NW, _SEQ_W, _L).astype(jnp.int32)
    pe_l = pe[0, :_L, :].astype(jnp.float32)
    return _emb_kernel(seq, seg, token_table, segment_table, pe_l)


# speedup vs baseline: 2.6340x; 1.2410x over previous
import functools

import jax
import jax.numpy as jnp
from jax import lax
from jax.experimental import pallas as pl
from jax.experimental.pallas import tpu as pltpu
from jax.experimental.pallas import tpu_sc as plsc

_B, _L, _E = 1024, 200, 64
_N = _B * _L
_NSEG = 17
_NC, _NS = 2, 16
_NW = _NC * _NS
_ROWS_W = _N // _NW
_SEQ_W = _ROWS_W // _L
_HL = _L // 2
_NBUF = 4

_mesh = plsc.VectorSubcoreMesh(core_axis_name="c", subcore_axis_name="s")


@functools.partial(
    pl.kernel,
    mesh=_mesh,
    out_type=jax.ShapeDtypeStruct((_B, _L, 2 * _E), jnp.float32),
    scratch_types=[
        pltpu.VMEM((2 * _SEQ_W, _HL), jnp.int32),
        pltpu.VMEM((_SEQ_W, _L), jnp.int32),
        pltpu.VMEM((_NSEG, _E), jnp.float32),
        [pltpu.VMEM((_L, _E), jnp.float32) for _ in range(_NBUF)],
        [pltpu.SemaphoreType.DMA for _ in range(3 * _NBUF)],
    ],
    compiler_params=pltpu.CompilerParams(use_tc_tiling_on_sc=False),
)
def _emb_kernel(seq_hbm, seglab_hbm, tok_tab, seg_tab, pe_hbm, out_hbm,
                tokidx, seglab, segtab_v, outb, sems):
    wid = lax.axis_index("s") * _NC + lax.axis_index("c")
    base = wid * _ROWS_W
    sem_p, sem_a, sem_sc = sems[0:_NBUF], sems[_NBUF:2 * _NBUF], sems[2 * _NBUF:]

    pltpu.sync_copy(seq_hbm.at[wid], tokidx)
    pltpu.sync_copy(seglab_hbm.at[wid], seglab)
    pltpu.sync_copy(seg_tab, segtab_v)

    def prefill(s, b):
        return pltpu.make_async_copy(pe_hbm, outb[b], sem_p[b])

    def adds(s, b):
        out = []
        for h in range(2):
            half = outb[b].at[pl.ds(h * _HL, _HL)]
            out.append(pltpu.make_async_copy(
                tok_tab.at[tokidx.at[2 * s + h]], half, sem_a[b]))
        return out

    def scatter(s, b):
        return pltpu.make_async_copy(
            outb[b], out_hbm.at[wid * _SEQ_W + s, :, pl.ds(0, _E)], sem_sc[b])

    def seg_add(s, b):
        ob = outb[b]

        def rows16(r0, labs, lo=0):
            for k in range(lo, 16):
                lab = labs[k]
                r = r0 + k
                for p2 in range(_E // 16):
                    sl = pl.ds(p2 * 16, 16)
                    ob[r, sl] = ob[r, sl] + segtab_v[lab, sl]

        def grp_body(i, c2):
            r0 = i * 16
            rows16(r0, seglab[s, pl.ds(r0, 16)])
            return c2

        lax.fori_loop(0, _L // 16, grp_body, 0)
        rows16(184, seglab[s, pl.ds(184, 16)], lo=8)

    def start_adds(s, b):
        for h in adds(s, b):
            h.start(add=True)

    def wait_adds(s, b):
        for h in adds(s, b):
            h.wait()

    prefill(0, 0).start()
    prefill(1, 1).start()
    prefill(0, 0).wait()
    start_adds(0, 0)
    prefill(2, 2).start()
    prefill(1, 1).wait()
    start_adds(1, 1)
    for s in (2, 3):
        t = s - 2
        wait_adds(t, t % 4)
        seg_add(t, t % 4)
        scatter(t, t % 4).start()
        if s == 3:
            scatter(0, 0).wait()
        prefill(s + 1, (s + 1) % 4).start()
        prefill(s, s % 4).wait()
        start_adds(s, s % 4)

    def steady(u, carry):
        for b in range(4):
            s = 4 * u + b
            t = s - 2
            wait_adds(t, (b - 2) % 4)
            seg_add(t, (b - 2) % 4)
            scatter(t, (b - 2) % 4).start()
            scatter(s - 3, (b + 1) % 4).wait()

            @pl.when(s < _SEQ_W - 1)
            def _():
                prefill(s + 1, (b + 1) % 4).start()

            prefill(s, b).wait()
            start_adds(s, b)
        return carry

    lax.fori_loop(1, _SEQ_W // 4, steady, 0)

    for s in (_SEQ_W, _SEQ_W + 1):
        t = s - 2
        wait_adds(t, t % 4)
        seg_add(t, t % 4)
        scatter(t, t % 4).start()
    for t in (_SEQ_W - 3, _SEQ_W - 2, _SEQ_W - 1):
        scatter(t, t % 4).wait()


def kernel(sequence, segment_label, token_table, segment_table, pe):
    seq = sequence.reshape(_NW, 2 * _SEQ_W, _HL).astype(jnp.int32)
    seg = segment_label.reshape(_NW, _SEQ_W, _L).astype(jnp.int32)
    pe_l = pe[0, :_L, :].astype(jnp.float32)
    out = _emb_kernel(seq, seg, token_table, segment_table, pe_l)
    return out[:, :, :_E]
